# baseline (speedup 1.0000x reference)
import functools

import jax
import jax.numpy as jnp
from jax import lax
from jax.experimental import pallas as pl
from jax.experimental.pallas import tpu as pltpu
from jax.experimental.pallas import tpu_sc as plsc

K = 8
SC_SLABS = 64


SC_CHUNK = 4


def _sc_partial_kernel(x_hbm, out_hbm, buf0, buf1, acc, sem0, sem1,
                       *, s0, s1, bW, dim):
    wid = lax.axis_index("s") * 2 + lax.axis_index("c")
    base = wid * bW
    bufs = (buf0, buf1)
    sems = (sem0, sem1)
    nch = (s1 - s0) // SC_CHUNK

    def start_fetch(c, b):
        pltpu.make_async_copy(
            x_hbm.at[pl.ds(s0 + c * SC_CHUNK, SC_CHUNK), pl.ds(base, bW), :],
            bufs[b], sems[b]).start()

    start_fetch(0, 0)
    start_fetch(1, 1)

    zero = jnp.zeros((16,), jnp.float32)

    @pl.loop(0, bW)
    def _zero_row(r):
        for j in range(dim // 16):
            acc[r, pl.ds(j * 16, 16)] = zero

    @pl.loop(0, nch, step=2)
    def _chunk(c):
        for b in range(2):
            cc = c + b
            pltpu.make_async_copy(
                x_hbm.at[pl.ds(s0 + cc * SC_CHUNK, SC_CHUNK),
                         pl.ds(base, bW), :],
                bufs[b], sems[b]).wait()

            @pl.loop(0, bW, unroll=4)
            def _row(r):
                for s in range(SC_CHUNK):
                    for j in range(dim // 16):
                        sl = pl.ds(j * 16, 16)
                        plsc.addupdate(acc.at[r, sl], bufs[b][s, r, sl])

            @pl.when(cc + 2 < nch)
            def _start_next():
                start_fetch(cc + 2, b)

    pltpu.sync_copy(acc, out_hbm.at[pl.ds(base, bW), :])


def _tc_reduce_kernel(x_ref, psum_ref):
    psum_ref[...] = jnp.sum(x_ref[...], axis=0)


def _router_kernel(tcp_ref, scp_ref, freq_ref, noise_ref, gw_ref, fgw_ref,
                   gates_ref, idx_ref, val_ref, *, spatial):
    pooled = (tcp_ref[...] + scp_ref[...]) * (1.0 / spatial)
    logits = (
        jax.lax.dot(pooled, gw_ref[...], preferred_element_type=jnp.float32)
        + jax.lax.dot(freq_ref[...], fgw_ref[...],
                      preferred_element_type=jnp.float32)
        + noise_ref[...]
    )
    m = jnp.max(logits, axis=1, keepdims=True)
    e = jnp.exp(logits - m)
    probs = e / jnp.sum(e, axis=1, keepdims=True)

    bB, E = probs.shape
    lane = jax.lax.broadcasted_iota(jnp.int32, (bB, E), 1)
    work = probs
    keep = jnp.zeros((bB, E), dtype=jnp.bool_)
    vals = []
    idxs = []
    for _ in range(K):
        cur = jnp.max(work, axis=1, keepdims=True)
        cur_i = jnp.min(jnp.where(work == cur, lane, E), axis=1,
                        keepdims=True)
        sel = lane == cur_i
        keep = jnp.logical_or(keep, sel)
        work = jnp.where(sel, -jnp.inf, work)
        vals.append(cur)
        idxs.append(cur_i)
    gates_ref[...] = jnp.where(keep, probs, 0.0)
    val_ref[...] = jnp.concatenate(vals, axis=1)
    idx_ref[...] = jnp.concatenate(idxs, axis=1)


def kernel(x, freq_emb, gate_w, freq_gate_w):
    B, DIM, H, W = x.shape
    FREQ = freq_emb.shape[1]
    E = gate_w.shape[0]
    S = H * W
    S_TC = S - SC_SLABS
    noise_std = 1.0 / E
    noise = jax.random.normal(jax.random.key(42), (B, E),
                              dtype=jnp.float32) * noise_std

    x_t = x.transpose(2, 3, 0, 1).reshape(S, B, DIM)
    gw_t = gate_w.T
    fgw_t = freq_gate_w.T

    bW = B // 32
    sc_partial = pl.kernel(
        functools.partial(_sc_partial_kernel, s0=S_TC, s1=S, bW=bW, dim=DIM),
        out_type=jax.ShapeDtypeStruct((B, DIM), jnp.float32),
        mesh=plsc.VectorSubcoreMesh(core_axis_name="c", subcore_axis_name="s"),
        scratch_types=[
            pltpu.VMEM((SC_CHUNK, bW, DIM), jnp.float32),
            pltpu.VMEM((SC_CHUNK, bW, DIM), jnp.float32),
            pltpu.VMEM((bW, DIM), jnp.float32),
            pltpu.SemaphoreType.DMA,
            pltpu.SemaphoreType.DMA,
        ],
    )(x_t)

    bB = 128
    tc_partial = pl.pallas_call(
        _tc_reduce_kernel,
        grid=(B // bB,),
        in_specs=[pl.BlockSpec((S_TC, bB, DIM), lambda i: (0, i, 0))],
        out_specs=pl.BlockSpec((bB, DIM), lambda i: (i, 0)),
        out_shape=jax.ShapeDtypeStruct((B, DIM), jnp.float32),
        compiler_params=pltpu.CompilerParams(
            dimension_semantics=("arbitrary",),
        ),
    )(x_t)

    bB2 = 256
    gates, idxs, vals = pl.pallas_call(
        functools.partial(_router_kernel, spatial=float(S)),
        grid=(B // bB2,),
        in_specs=[
            pl.BlockSpec((bB2, DIM), lambda i: (i, 0)),
            pl.BlockSpec((bB2, DIM), lambda i: (i, 0)),
            pl.BlockSpec((bB2, FREQ), lambda i: (i, 0)),
            pl.BlockSpec((bB2, E), lambda i: (i, 0)),
            pl.BlockSpec((DIM, E), lambda i: (0, 0)),
            pl.BlockSpec((FREQ, E), lambda i: (0, 0)),
        ],
        out_specs=[
            pl.BlockSpec((bB2, E), lambda i: (i, 0)),
            pl.BlockSpec((bB2, K), lambda i: (i, 0)),
            pl.BlockSpec((bB2, K), lambda i: (i, 0)),
        ],
        out_shape=[
            jax.ShapeDtypeStruct((B, E), jnp.float32),
            jax.ShapeDtypeStruct((B, K), jnp.int32),
            jax.ShapeDtypeStruct((B, K), jnp.float32),
        ],
        compiler_params=pltpu.CompilerParams(
            dimension_semantics=("arbitrary",),
        ),
    )(tc_partial, sc_partial, freq_emb, noise, gw_t, fgw_t)

    return (gates, idxs, vals, jnp.float32(0.0))


# speedup vs baseline: 1.4802x; 1.4802x over previous
import functools

import jax
import jax.numpy as jnp
from jax import lax
from jax.experimental import pallas as pl
from jax.experimental.pallas import tpu as pltpu
from jax.experimental.pallas import tpu_sc as plsc

K = 8
SC_SLABS = 64


SC_CHUNK = 4


def _sc_partial_kernel(x_hbm, out_hbm, buf0, buf1, acc, sem0, sem1,
                       *, s0, s1, bW, dim):
    wid = lax.axis_index("s") * 2 + lax.axis_index("c")
    base = wid * bW
    bufs = (buf0, buf1)
    sems = (sem0, sem1)
    nch = (s1 - s0) // SC_CHUNK

    def start_fetch(c, b):
        pltpu.make_async_copy(
            x_hbm.at[pl.ds(s0 + c * SC_CHUNK, SC_CHUNK), pl.ds(base, bW), :],
            bufs[b], sems[b]).start()

    start_fetch(0, 0)
    start_fetch(1, 1)

    zero = jnp.zeros((16,), jnp.float32)

    @pl.loop(0, bW)
    def _zero_row(r):
        for j in range(dim // 16):
            acc[r, pl.ds(j * 16, 16)] = zero

    @pl.loop(0, nch, step=2)
    def _chunk(c):
        for b in range(2):
            cc = c + b
            pltpu.make_async_copy(
                x_hbm.at[pl.ds(s0 + cc * SC_CHUNK, SC_CHUNK),
                         pl.ds(base, bW), :],
                bufs[b], sems[b]).wait()

            @pl.loop(0, bW)
            def _row(r):
                for j in range(dim // 16):
                    sl = pl.ds(j * 16, 16)
                    v0 = bufs[b][0, r, sl]
                    v1 = bufs[b][1, r, sl]
                    v2 = bufs[b][2, r, sl]
                    v3 = bufs[b][3, r, sl]
                    plsc.addupdate(acc.at[r, sl], (v0 + v1) + (v2 + v3))

            @pl.when(cc + 2 < nch)
            def _start_next():
                start_fetch(cc + 2, b)

    pltpu.sync_copy(acc, out_hbm.at[pl.ds(base, bW), :])


def _tc_reduce_kernel(x_ref, psum_ref):
    psum_ref[...] = jnp.sum(x_ref[...], axis=0)


def _router_kernel(tcp_ref, scp_ref, freq_ref, noise_ref, gw_ref, fgw_ref,
                   gates_ref, idx_ref, val_ref, *, spatial):
    pooled = (tcp_ref[...] + scp_ref[...]) * (1.0 / spatial)
    logits = (
        jax.lax.dot(pooled, gw_ref[...], preferred_element_type=jnp.float32)
        + jax.lax.dot(freq_ref[...], fgw_ref[...],
                      preferred_element_type=jnp.float32)
        + noise_ref[...]
    )
    m = jnp.max(logits, axis=1, keepdims=True)
    e = jnp.exp(logits - m)
    probs = e / jnp.sum(e, axis=1, keepdims=True)

    bB, E = probs.shape
    lane = jax.lax.broadcasted_iota(jnp.int32, (bB, E), 1)
    work = probs
    keep = jnp.zeros((bB, E), dtype=jnp.bool_)
    vals = []
    idxs = []
    for _ in range(K):
        cur = jnp.max(work, axis=1, keepdims=True)
        cur_i = jnp.min(jnp.where(work == cur, lane, E), axis=1,
                        keepdims=True)
        sel = lane == cur_i
        keep = jnp.logical_or(keep, sel)
        work = jnp.where(sel, -jnp.inf, work)
        vals.append(cur)
        idxs.append(cur_i)
    gates_ref[...] = jnp.where(keep, probs, 0.0)
    val_ref[...] = jnp.concatenate(vals, axis=1)
    idx_ref[...] = jnp.concatenate(idxs, axis=1)


def kernel(x, freq_emb, gate_w, freq_gate_w):
    B, DIM, H, W = x.shape
    FREQ = freq_emb.shape[1]
    E = gate_w.shape[0]
    S = H * W
    S_TC = S - SC_SLABS
    noise_std = 1.0 / E
    noise = jax.random.normal(jax.random.key(42), (B, E),
                              dtype=jnp.float32) * noise_std

    x_t = x.transpose(2, 3, 0, 1).reshape(S, B, DIM)
    gw_t = gate_w.T
    fgw_t = freq_gate_w.T

    bW = B // 32
    sc_partial = pl.kernel(
        functools.partial(_sc_partial_kernel, s0=S_TC, s1=S, bW=bW, dim=DIM),
        out_type=jax.ShapeDtypeStruct((B, DIM), jnp.float32),
        mesh=plsc.VectorSubcoreMesh(core_axis_name="c", subcore_axis_name="s"),
        scratch_types=[
            pltpu.VMEM((SC_CHUNK, bW, DIM), jnp.float32),
            pltpu.VMEM((SC_CHUNK, bW, DIM), jnp.float32),
            pltpu.VMEM((bW, DIM), jnp.float32),
            pltpu.SemaphoreType.DMA,
            pltpu.SemaphoreType.DMA,
        ],
    )(x_t)

    bB = 128
    tc_partial = pl.pallas_call(
        _tc_reduce_kernel,
        grid=(B // bB,),
        in_specs=[pl.BlockSpec((S_TC, bB, DIM), lambda i: (0, i, 0))],
        out_specs=pl.BlockSpec((bB, DIM), lambda i: (i, 0)),
        out_shape=jax.ShapeDtypeStruct((B, DIM), jnp.float32),
        compiler_params=pltpu.CompilerParams(
            dimension_semantics=("arbitrary",),
        ),
    )(x_t)

    bB2 = 256
    gates, idxs, vals = pl.pallas_call(
        functools.partial(_router_kernel, spatial=float(S)),
        grid=(B // bB2,),
        in_specs=[
            pl.BlockSpec((bB2, DIM), lambda i: (i, 0)),
            pl.BlockSpec((bB2, DIM), lambda i: (i, 0)),
            pl.BlockSpec((bB2, FREQ), lambda i: (i, 0)),
            pl.BlockSpec((bB2, E), lambda i: (i, 0)),
            pl.BlockSpec((DIM, E), lambda i: (0, 0)),
            pl.BlockSpec((FREQ, E), lambda i: (0, 0)),
        ],
        out_specs=[
            pl.BlockSpec((bB2, E), lambda i: (i, 0)),
            pl.BlockSpec((bB2, K), lambda i: (i, 0)),
            pl.BlockSpec((bB2, K), lambda i: (i, 0)),
        ],
        out_shape=[
            jax.ShapeDtypeStruct((B, E), jnp.float32),
            jax.ShapeDtypeStruct((B, K), jnp.int32),
            jax.ShapeDtypeStruct((B, K), jnp.float32),
        ],
        compiler_params=pltpu.CompilerParams(
            dimension_semantics=("arbitrary",),
        ),
    )(tc_partial, sc_partial, freq_emb, noise, gw_t, fgw_t)

    return (gates, idxs, vals, jnp.float32(0.0))


# speedup vs baseline: 1.5918x; 1.0754x over previous
import functools

import jax
import jax.numpy as jnp
from jax import lax
from jax.experimental import pallas as pl
from jax.experimental.pallas import tpu as pltpu
from jax.experimental.pallas import tpu_sc as plsc

K = 8
SC_BATCH = 256
SC_CHUNK = 14


def _sc_batch_kernel(x_hbm, out_hbm, buf0, buf1, acc, sem0, sem1,
                     *, b0, bW, dim, spatial):
    wid = lax.axis_index("s") * 2 + lax.axis_index("c")
    base = b0 + wid * bW
    bufs = (buf0, buf1)
    sems = (sem0, sem1)
    nch = spatial // SC_CHUNK

    def start_fetch(c, b):
        pltpu.make_async_copy(
            x_hbm.at[pl.ds(c * SC_CHUNK, SC_CHUNK), pl.ds(base, bW), :],
            bufs[b], sems[b]).start()

    start_fetch(0, 0)
    start_fetch(1, 1)

    zero = jnp.zeros((16,), jnp.float32)

    @pl.loop(0, bW)
    def _zero_row(r):
        for j in range(dim // 16):
            acc[r, pl.ds(j * 16, 16)] = zero

    @pl.loop(0, nch, step=2)
    def _chunk(c):
        for b in range(2):
            cc = c + b
            pltpu.make_async_copy(
                x_hbm.at[pl.ds(cc * SC_CHUNK, SC_CHUNK), pl.ds(base, bW), :],
                bufs[b], sems[b]).wait()

            @pl.loop(0, bW)
            def _row(r):
                for j in range(dim // 16):
                    sl = pl.ds(j * 16, 16)
                    vs = [bufs[b][s, r, sl] for s in range(SC_CHUNK)]
                    while len(vs) > 1:
                        vs = [vs[i] + vs[i + 1] for i in range(0, len(vs) - 1, 2)] \
                             + ([vs[-1]] if len(vs) % 2 else [])
                    plsc.addupdate(acc.at[r, sl], vs[0])

            @pl.when(cc + 2 < nch)
            def _start_next():
                start_fetch(cc + 2, b)

    pltpu.sync_copy(acc, out_hbm.at[pl.ds(wid * bW, bW), :])


def _epilogue(pooled, freq, noise, gw, fgw):
    logits = (
        jax.lax.dot(pooled, gw, preferred_element_type=jnp.float32)
        + jax.lax.dot(freq, fgw, preferred_element_type=jnp.float32)
        + noise
    )
    m = jnp.max(logits, axis=1, keepdims=True)
    e = jnp.exp(logits - m)
    probs = e / jnp.sum(e, axis=1, keepdims=True)

    bB, E = probs.shape
    lane = jax.lax.broadcasted_iota(jnp.int32, (bB, E), 1)
    work = probs
    keep = jnp.zeros((bB, E), dtype=jnp.bool_)
    vals = []
    idxs = []
    for _ in range(K):
        cur = jnp.max(work, axis=1, keepdims=True)
        cur_i = jnp.min(jnp.where(work == cur, lane, E), axis=1,
                        keepdims=True)
        sel = lane == cur_i
        keep = jnp.logical_or(keep, sel)
        work = jnp.where(sel, -jnp.inf, work)
        vals.append(cur)
        idxs.append(cur_i)
    gates = jnp.where(keep, probs, 0.0)
    return gates, jnp.concatenate(idxs, axis=1), jnp.concatenate(vals, axis=1)


def _tc_fused_kernel(x_ref, freq_ref, noise_ref, gw_ref, fgw_ref,
                     gates_ref, idx_ref, val_ref, *, spatial):
    pooled = jnp.sum(x_ref[...], axis=0) * (1.0 / spatial)
    g, i, v = _epilogue(pooled, freq_ref[...], noise_ref[...],
                        gw_ref[...], fgw_ref[...])
    gates_ref[...] = g
    idx_ref[...] = i
    val_ref[...] = v


def _tc_tail_kernel(scp_ref, freq_ref, noise_ref, gw_ref, fgw_ref,
                    gates_ref, idx_ref, val_ref, *, spatial):
    pooled = scp_ref[...] * (1.0 / spatial)
    g, i, v = _epilogue(pooled, freq_ref[...], noise_ref[...],
                        gw_ref[...], fgw_ref[...])
    gates_ref[...] = g
    idx_ref[...] = i
    val_ref[...] = v


def kernel(x, freq_emb, gate_w, freq_gate_w):
    B, DIM, H, W = x.shape
    FREQ = freq_emb.shape[1]
    E = gate_w.shape[0]
    S = H * W
    B_TC = B - SC_BATCH
    noise_std = 1.0 / E
    noise = jax.random.normal(jax.random.key(42), (B, E),
                              dtype=jnp.float32) * noise_std

    x_t = x.transpose(2, 3, 0, 1).reshape(S, B, DIM)
    gw_t = gate_w.T
    fgw_t = freq_gate_w.T

    bW = SC_BATCH // 32
    sc_partial = pl.kernel(
        functools.partial(_sc_batch_kernel, b0=B_TC, bW=bW, dim=DIM,
                          spatial=S),
        out_type=jax.ShapeDtypeStruct((SC_BATCH, DIM), jnp.float32),
        mesh=plsc.VectorSubcoreMesh(core_axis_name="c", subcore_axis_name="s"),
        scratch_types=[
            pltpu.VMEM((SC_CHUNK, bW, DIM), jnp.float32),
            pltpu.VMEM((SC_CHUNK, bW, DIM), jnp.float32),
            pltpu.VMEM((bW, DIM), jnp.float32),
            pltpu.SemaphoreType.DMA,
            pltpu.SemaphoreType.DMA,
        ],
    )(x_t)

    bB = 128
    tc_outs = pl.pallas_call(
        functools.partial(_tc_fused_kernel, spatial=float(S)),
        grid=(B_TC // bB,),
        in_specs=[
            pl.BlockSpec((S, bB, DIM), lambda i: (0, i, 0)),
            pl.BlockSpec((bB, FREQ), lambda i: (i, 0)),
            pl.BlockSpec((bB, E), lambda i: (i, 0)),
            pl.BlockSpec((DIM, E), lambda i: (0, 0)),
            pl.BlockSpec((FREQ, E), lambda i: (0, 0)),
        ],
        out_specs=[
            pl.BlockSpec((bB, E), lambda i: (i, 0)),
            pl.BlockSpec((bB, K), lambda i: (i, 0)),
            pl.BlockSpec((bB, K), lambda i: (i, 0)),
        ],
        out_shape=[
            jax.ShapeDtypeStruct((B_TC, E), jnp.float32),
            jax.ShapeDtypeStruct((B_TC, K), jnp.int32),
            jax.ShapeDtypeStruct((B_TC, K), jnp.float32),
        ],
        compiler_params=pltpu.CompilerParams(
            dimension_semantics=("arbitrary",),
        ),
    )(x_t, freq_emb, noise, gw_t, fgw_t)

    nb_tail = B // SC_BATCH - 1
    sc_outs = pl.pallas_call(
        functools.partial(_tc_tail_kernel, spatial=float(S)),
        grid=(1,),
        in_specs=[
            pl.BlockSpec((SC_BATCH, DIM), lambda i: (0, 0)),
            pl.BlockSpec((SC_BATCH, FREQ), lambda i: (nb_tail, 0)),
            pl.BlockSpec((SC_BATCH, E), lambda i: (nb_tail, 0)),
            pl.BlockSpec((DIM, E), lambda i: (0, 0)),
            pl.BlockSpec((FREQ, E), lambda i: (0, 0)),
        ],
        out_specs=[
            pl.BlockSpec((SC_BATCH, E), lambda i: (0, 0)),
            pl.BlockSpec((SC_BATCH, K), lambda i: (0, 0)),
            pl.BlockSpec((SC_BATCH, K), lambda i: (0, 0)),
        ],
        out_shape=[
            jax.ShapeDtypeStruct((SC_BATCH, E), jnp.float32),
            jax.ShapeDtypeStruct((SC_BATCH, K), jnp.int32),
            jax.ShapeDtypeStruct((SC_BATCH, K), jnp.float32),
        ],
        compiler_params=pltpu.CompilerParams(
            dimension_semantics=("arbitrary",),
        ),
    )(sc_partial, freq_emb, noise, gw_t, fgw_t)

    gates = jnp.concatenate([tc_outs[0], sc_outs[0]], axis=0)
    idxs = jnp.concatenate([tc_outs[1], sc_outs[1]], axis=0)
    vals = jnp.concatenate([tc_outs[2], sc_outs[2]], axis=0)
    return (gates, idxs, vals, jnp.float32(0.0))


# speedup vs baseline: 1.6251x; 1.0209x over previous
import functools

import jax
import jax.numpy as jnp
from jax import lax
from jax.experimental import pallas as pl
from jax.experimental.pallas import tpu as pltpu
from jax.experimental.pallas import tpu_sc as plsc

K = 8
SC_BATCH = 384
SC_CHUNK = 14


def _sc_batch_kernel(x_hbm, out_hbm, buf0, buf1, acc, sem0, sem1,
                     *, b0, bW, dim, spatial):
    wid = lax.axis_index("s") * 2 + lax.axis_index("c")
    bufs = (buf0, buf1)
    sems = (sem0, sem1)

    zero = jnp.zeros((16,), jnp.float32)

    def sum_region(base, s0, chunk, nch, out_row):
        def start_fetch(c, b):
            pltpu.make_async_copy(
                x_hbm.at[pl.ds(s0 + c * chunk, chunk), pl.ds(base, bW), :],
                bufs[b].at[pl.ds(0, chunk)], sems[b]).start()

        start_fetch(0, 0)
        start_fetch(1, 1)

        @pl.loop(0, bW)
        def _zero_row(r):
            for j in range(dim // 16):
                acc[r, pl.ds(j * 16, 16)] = zero

        @pl.loop(0, nch, step=2)
        def _chunk(c):
            for b in range(2):
                cc = c + b
                pltpu.make_async_copy(
                    x_hbm.at[pl.ds(s0 + cc * chunk, chunk),
                             pl.ds(base, bW), :],
                    bufs[b].at[pl.ds(0, chunk)], sems[b]).wait()

                @pl.loop(0, bW)
                def _row(r):
                    for j in range(dim // 16):
                        sl = pl.ds(j * 16, 16)
                        vs = [bufs[b][s, r, sl] for s in range(chunk)]
                        while len(vs) > 1:
                            vs = [vs[i] + vs[i + 1]
                                  for i in range(0, len(vs) - 1, 2)] \
                                 + ([vs[-1]] if len(vs) % 2 else [])
                        plsc.addupdate(acc.at[r, sl], vs[0])

                @pl.when(cc + 2 < nch)
                def _start_next():
                    start_fetch(cc + 2, b)

        pltpu.sync_copy(acc, out_hbm.at[pl.ds(out_row, bW), :])

    sum_region(b0 + wid * bW, 0, SC_CHUNK, spatial // SC_CHUNK, wid * bW)
    half = spatial // 2
    grp = wid // 2
    hlf = wid % 2
    sum_region(b0 + 256 + grp * bW, hlf * half, SC_CHUNK // 2,
               half // (SC_CHUNK // 2), 256 + hlf * 128 + grp * bW)


def _epilogue(pooled, freq, noise, gw, fgw):
    dn = (((1,), (1,)), ((), ()))
    logits = (
        lax.dot_general(pooled, gw, dn, preferred_element_type=jnp.float32)
        + lax.dot_general(freq, fgw, dn, preferred_element_type=jnp.float32)
        + noise
    )
    m = jnp.max(logits, axis=1, keepdims=True)
    e = jnp.exp(logits - m)
    probs = e / jnp.sum(e, axis=1, keepdims=True)

    bB, E = probs.shape
    lane = jax.lax.broadcasted_iota(jnp.int32, (bB, E), 1)
    work = probs
    keep = jnp.zeros((bB, E), dtype=jnp.bool_)
    vals = []
    idxs = []
    for _ in range(K):
        cur = jnp.max(work, axis=1, keepdims=True)
        cur_i = jnp.min(jnp.where(work == cur, lane, E), axis=1,
                        keepdims=True)
        sel = lane == cur_i
        keep = jnp.logical_or(keep, sel)
        work = jnp.where(sel, -jnp.inf, work)
        vals.append(cur)
        idxs.append(cur_i)
    gates = jnp.where(keep, probs, 0.0)
    return gates, jnp.concatenate(idxs, axis=1), jnp.concatenate(vals, axis=1)


def _tc_fused_kernel(x_ref, freq_ref, noise_ref, gw_ref, fgw_ref,
                     gates_ref, idx_ref, val_ref, *, spatial):
    pooled = jnp.sum(x_ref[...], axis=0) * (1.0 / spatial)
    g, i, v = _epilogue(pooled, freq_ref[...], noise_ref[...],
                        gw_ref[...], fgw_ref[...])
    gates_ref[...] = g
    idx_ref[...] = i
    val_ref[...] = v


def _tc_tail_kernel(g_in, i_in, v_in, scp_ref, freq_ref, noise_ref,
                    gw_ref, fgw_ref, gates_ref, idx_ref, val_ref,
                    *, spatial):
    del g_in, i_in, v_in
    pid = pl.program_id(0)
    d = scp_ref[pl.ds(pid * 128, 128), :]
    h1 = scp_ref[pl.ds(384, 128), :]
    pooled = jnp.where(pid == 2, d + h1, d) * (1.0 / spatial)
    g, i, v = _epilogue(pooled, freq_ref[...], noise_ref[...],
                        gw_ref[...], fgw_ref[...])
    gates_ref[...] = g
    idx_ref[...] = i
    val_ref[...] = v


def kernel(x, freq_emb, gate_w, freq_gate_w):
    B, DIM, H, W = x.shape
    FREQ = freq_emb.shape[1]
    E = gate_w.shape[0]
    S = H * W
    B_TC = B - SC_BATCH
    bB = 128
    nb_tc = B_TC // bB

    with jax.ensure_compile_time_eval():
        noise = jax.random.normal(jax.random.key(42), (B, E),
                                  dtype=jnp.float32) * (1.0 / E)

    x_t = x.transpose(2, 3, 0, 1).reshape(S, B, DIM)

    bW = 8
    sc_partial = pl.kernel(
        functools.partial(_sc_batch_kernel, b0=B_TC, bW=bW, dim=DIM,
                          spatial=S),
        out_type=jax.ShapeDtypeStruct((SC_BATCH + 128, DIM), jnp.float32),
        mesh=plsc.VectorSubcoreMesh(core_axis_name="c", subcore_axis_name="s"),
        scratch_types=[
            pltpu.VMEM((SC_CHUNK, bW, DIM), jnp.float32),
            pltpu.VMEM((SC_CHUNK, bW, DIM), jnp.float32),
            pltpu.VMEM((bW, DIM), jnp.float32),
            pltpu.SemaphoreType.DMA,
            pltpu.SemaphoreType.DMA,
        ],
    )(x_t)

    tc_outs = pl.pallas_call(
        functools.partial(_tc_fused_kernel, spatial=float(S)),
        grid=(nb_tc,),
        in_specs=[
            pl.BlockSpec((S, bB, DIM), lambda i: (0, i, 0)),
            pl.BlockSpec((bB, FREQ), lambda i: (i, 0)),
            pl.BlockSpec((bB, E), lambda i: (i, 0)),
            pl.BlockSpec((E, DIM), lambda i: (0, 0)),
            pl.BlockSpec((E, FREQ), lambda i: (0, 0)),
        ],
        out_specs=[
            pl.BlockSpec((bB, E), lambda i: (i, 0)),
            pl.BlockSpec((bB, K), lambda i: (i, 0)),
            pl.BlockSpec((bB, K), lambda i: (i, 0)),
        ],
        out_shape=[
            jax.ShapeDtypeStruct((B, E), jnp.float32),
            jax.ShapeDtypeStruct((B, K), jnp.int32),
            jax.ShapeDtypeStruct((B, K), jnp.float32),
        ],
        compiler_params=pltpu.CompilerParams(
            dimension_semantics=("arbitrary",),
        ),
    )(x_t, freq_emb, noise, gate_w, freq_gate_w)

    outs = pl.pallas_call(
        functools.partial(_tc_tail_kernel, spatial=float(S)),
        grid=(SC_BATCH // bB,),
        in_specs=[
            pl.BlockSpec(memory_space=pl.ANY),
            pl.BlockSpec(memory_space=pl.ANY),
            pl.BlockSpec(memory_space=pl.ANY),
            pl.BlockSpec((SC_BATCH + 128, DIM), lambda i: (0, 0)),
            pl.BlockSpec((bB, FREQ), lambda i: (nb_tc + i, 0)),
            pl.BlockSpec((bB, E), lambda i: (nb_tc + i, 0)),
            pl.BlockSpec((E, DIM), lambda i: (0, 0)),
            pl.BlockSpec((E, FREQ), lambda i: (0, 0)),
        ],
        out_specs=[
            pl.BlockSpec((bB, E), lambda i: (nb_tc + i, 0)),
            pl.BlockSpec((bB, K), lambda i: (nb_tc + i, 0)),
            pl.BlockSpec((bB, K), lambda i: (nb_tc + i, 0)),
        ],
        out_shape=[
            jax.ShapeDtypeStruct((B, E), jnp.float32),
            jax.ShapeDtypeStruct((B, K), jnp.int32),
            jax.ShapeDtypeStruct((B, K), jnp.float32),
        ],
        input_output_aliases={0: 0, 1: 1, 2: 2},
        compiler_params=pltpu.CompilerParams(
            dimension_semantics=("arbitrary",),
        ),
    )(tc_outs[0], tc_outs[1], tc_outs[2], sc_partial, freq_emb, noise,
      gate_w, freq_gate_w)

    return (outs[0], outs[1], outs[2], jnp.float32(0.0))
